# baseline (device time: 57740 ns/iter reference)
import jax
import jax.numpy as jnp
from jax import lax
from jax.experimental import pallas as pl
from jax.experimental.pallas import tpu as pltpu

N_DEV = 4
N_LAYERS = 3


def kernel(x, Win0, Wout0, Win1, Wout1, Win2, Wout2):
    b, d_sh = x.shape
    h_dim = Win0.shape[1]

    xb = x.astype(jnp.bfloat16)
    w0i = Win0.astype(jnp.bfloat16)
    w0o = Wout0.astype(jnp.bfloat16)
    w1i = Win1.astype(jnp.bfloat16)
    w1o = Wout1.astype(jnp.bfloat16)
    w2i = Win2.astype(jnp.bfloat16)
    w2o = Wout2.astype(jnp.bfloat16)

    def body(x_ref, win0_ref, wout0_ref, win1_ref, wout1_ref,
             win2_ref, wout2_ref, out_ref,
             sbuf, rbuf, send_sems, recv_sems):
        my = lax.axis_index("i")
        p1 = my ^ 1
        p2 = 3 - my

        barrier = pltpu.get_barrier_semaphore()
        for nbr in (p1, p2):
            pl.semaphore_signal(
                barrier, inc=1,
                device_id=(nbr,), device_id_type=pl.DeviceIdType.MESH,
            )
        pl.semaphore_wait(barrier, 2)

        win_refs = [win0_ref, win1_ref, win2_ref]
        wout_refs = [wout0_ref, wout1_ref, wout2_ref]

        x_cur = x_ref[...]
        for l in range(N_LAYERS):
            ph = jnp.dot(x_cur, win_refs[l][...],
                         preferred_element_type=jnp.float32)

            sbuf[l, 0] = ph.astype(jnp.bfloat16)
            rdma1 = pltpu.make_async_remote_copy(
                src_ref=sbuf.at[l, 0],
                dst_ref=rbuf.at[l, 0],
                send_sem=send_sems.at[l, 0],
                recv_sem=recv_sems.at[l, 0],
                device_id=(p1,),
                device_id_type=pl.DeviceIdType.MESH,
            )
            rdma1.start()
            rdma1.wait()
            s1 = ph + rbuf[l, 0].astype(jnp.float32)

            sbuf[l, 1] = s1.astype(jnp.bfloat16)
            rdma2 = pltpu.make_async_remote_copy(
                src_ref=sbuf.at[l, 1],
                dst_ref=rbuf.at[l, 1],
                send_sem=send_sems.at[l, 1],
                recv_sem=recv_sems.at[l, 1],
                device_id=(p2,),
                device_id_type=pl.DeviceIdType.MESH,
            )
            rdma2.start()
            rdma2.wait()
            h_full = s1 + rbuf[l, 1].astype(jnp.float32)

            act = jnp.maximum(h_full, 0.0).astype(jnp.bfloat16)
            xn = jnp.dot(act, wout_refs[l][...],
                         preferred_element_type=jnp.float32)
            if l < N_LAYERS - 1:
                x_cur = xn.astype(jnp.bfloat16)
            else:
                out_ref[...] = xn

    return pl.pallas_call(
        body,
        out_shape=jax.ShapeDtypeStruct((b, d_sh), jnp.float32),
        in_specs=[pl.BlockSpec(memory_space=pltpu.VMEM)] * 7,
        out_specs=pl.BlockSpec(memory_space=pltpu.VMEM),
        scratch_shapes=[
            pltpu.VMEM((N_LAYERS, 2, b, h_dim), jnp.bfloat16),
            pltpu.VMEM((N_LAYERS, 2, b, h_dim), jnp.bfloat16),
            pltpu.SemaphoreType.DMA((N_LAYERS, 2)),
            pltpu.SemaphoreType.DMA((N_LAYERS, 2)),
        ],
        compiler_params=pltpu.CompilerParams(collective_id=0),
    )(xb, w0i, w0o, w1i, w1o, w2i, w2o)


# device time: 53175 ns/iter; 1.0858x vs baseline; 1.0858x over previous
import jax
import jax.numpy as jnp
from jax import lax
from jax.experimental import pallas as pl
from jax.experimental.pallas import tpu as pltpu

N_DEV = 4
N_LAYERS = 3


def kernel(x, Win0, Wout0, Win1, Wout1, Win2, Wout2):
    b, d_sh = x.shape
    h_dim = Win0.shape[1]

    def body(x_ref, win0_ref, wout0_ref, win1_ref, wout1_ref,
             win2_ref, wout2_ref, out_ref,
             sbuf, rbuf, send_sems, recv_sems):
        my = lax.axis_index("i")
        p1 = my ^ 1
        p2 = 3 - my

        barrier = pltpu.get_barrier_semaphore()
        for nbr in (p1, p2):
            pl.semaphore_signal(
                barrier, inc=1,
                device_id=(nbr,), device_id_type=pl.DeviceIdType.MESH,
            )
        pl.semaphore_wait(barrier, 2)

        win_refs = [win0_ref, win1_ref, win2_ref]
        wout_refs = [wout0_ref, wout1_ref, wout2_ref]

        x_cur = x_ref[...]
        for l in range(N_LAYERS):
            ph = jnp.dot(x_cur, win_refs[l][...],
                         preferred_element_type=jnp.float32)

            sbuf[l, 0] = ph.astype(jnp.bfloat16)
            rdma1 = pltpu.make_async_remote_copy(
                src_ref=sbuf.at[l, 0],
                dst_ref=rbuf.at[l, 0],
                send_sem=send_sems.at[l, 0],
                recv_sem=recv_sems.at[l, 0],
                device_id=(p1,),
                device_id_type=pl.DeviceIdType.MESH,
            )
            rdma1.start()
            rdma1.wait()
            s1 = ph + rbuf[l, 0].astype(jnp.float32)

            sbuf[l, 1] = s1.astype(jnp.bfloat16)
            rdma2 = pltpu.make_async_remote_copy(
                src_ref=sbuf.at[l, 1],
                dst_ref=rbuf.at[l, 1],
                send_sem=send_sems.at[l, 1],
                recv_sem=recv_sems.at[l, 1],
                device_id=(p2,),
                device_id_type=pl.DeviceIdType.MESH,
            )
            rdma2.start()
            rdma2.wait()
            h_full = s1 + rbuf[l, 1].astype(jnp.float32)

            act = jnp.maximum(h_full, 0.0)
            xn = jnp.dot(act, wout_refs[l][...],
                         preferred_element_type=jnp.float32)
            if l < N_LAYERS - 1:
                x_cur = xn
            else:
                out_ref[...] = xn

    return pl.pallas_call(
        body,
        out_shape=jax.ShapeDtypeStruct((b, d_sh), jnp.float32),
        in_specs=[pl.BlockSpec(memory_space=pltpu.VMEM)] * 7,
        out_specs=pl.BlockSpec(memory_space=pltpu.VMEM),
        scratch_shapes=[
            pltpu.VMEM((N_LAYERS, 2, b, h_dim), jnp.bfloat16),
            pltpu.VMEM((N_LAYERS, 2, b, h_dim), jnp.bfloat16),
            pltpu.SemaphoreType.DMA((N_LAYERS, 2)),
            pltpu.SemaphoreType.DMA((N_LAYERS, 2)),
        ],
        compiler_params=pltpu.CompilerParams(
            collective_id=0,
            vmem_limit_bytes=128 * 1024 * 1024,
        ),
    )(x, Win0, Wout0, Win1, Wout1, Win2, Wout2)


# device time: 42051 ns/iter; 1.3731x vs baseline; 1.2645x over previous
import jax
import jax.numpy as jnp
from jax import lax
from jax.experimental import pallas as pl
from jax.experimental.pallas import tpu as pltpu

N_DEV = 4
N_LAYERS = 3


def kernel(x, Win0, Wout0, Win1, Wout1, Win2, Wout2):
    b, d_sh = x.shape
    h_dim = Win0.shape[1]

    def body(x_ref, win0_ref, wout0_ref, win1_ref, wout1_ref,
             win2_ref, wout2_ref, out_ref,
             winf, woutf, winb, woutb, sbuf, rbuf,
             win_dsem, wout_dsem, send_sems, recv_sems):
        my = lax.axis_index("i")
        p1 = my ^ 1
        p2 = 3 - my

        win_hbm = [win0_ref, win1_ref, win2_ref]
        wout_hbm = [wout0_ref, wout1_ref, wout2_ref]
        win_cp = [
            pltpu.make_async_copy(win_hbm[l], winf.at[l % 2], win_dsem.at[l])
            for l in range(N_LAYERS)
        ]
        wout_cp = [
            pltpu.make_async_copy(wout_hbm[l], woutf.at[l % 2], wout_dsem.at[l])
            for l in range(N_LAYERS)
        ]

        win_cp[0].start()
        wout_cp[0].start()

        barrier = pltpu.get_barrier_semaphore()
        for nbr in (p1, p2):
            pl.semaphore_signal(
                barrier, inc=1,
                device_id=(nbr,), device_id_type=pl.DeviceIdType.MESH,
            )
        pl.semaphore_wait(barrier, 2)

        x_cur = x_ref[...].astype(jnp.bfloat16)

        win_cp[0].wait()
        winb[...] = winf[0].astype(jnp.bfloat16)
        win_cp[1].start()

        for l in range(N_LAYERS):
            ph = jnp.dot(x_cur, winb[...],
                         preferred_element_type=jnp.float32)

            sbuf[l, 0] = ph.astype(jnp.bfloat16)
            rdma1 = pltpu.make_async_remote_copy(
                src_ref=sbuf.at[l, 0],
                dst_ref=rbuf.at[l, 0],
                send_sem=send_sems.at[l, 0],
                recv_sem=recv_sems.at[l, 0],
                device_id=(p1,),
                device_id_type=pl.DeviceIdType.MESH,
            )
            rdma1.start()
            wout_cp[l].wait()
            woutb[...] = woutf[l % 2].astype(jnp.bfloat16)
            if l + 1 < N_LAYERS:
                wout_cp[l + 1].start()
            rdma1.wait()
            sbuf[l, 1] = (
                sbuf[l, 0].astype(jnp.float32)
                + rbuf[l, 0].astype(jnp.float32)
            ).astype(jnp.bfloat16)

            rdma2 = pltpu.make_async_remote_copy(
                src_ref=sbuf.at[l, 1],
                dst_ref=rbuf.at[l, 1],
                send_sem=send_sems.at[l, 1],
                recv_sem=recv_sems.at[l, 1],
                device_id=(p2,),
                device_id_type=pl.DeviceIdType.MESH,
            )
            rdma2.start()
            if l + 1 < N_LAYERS:
                win_cp[l + 1].wait()
                winb[...] = winf[(l + 1) % 2].astype(jnp.bfloat16)
                if l + 2 < N_LAYERS:
                    win_cp[l + 2].start()
            rdma2.wait()
            h_full = (
                sbuf[l, 1].astype(jnp.float32)
                + rbuf[l, 1].astype(jnp.float32)
            )

            act = jnp.maximum(h_full, 0.0).astype(jnp.bfloat16)
            xn = jnp.dot(act, woutb[...],
                         preferred_element_type=jnp.float32)
            if l < N_LAYERS - 1:
                x_cur = xn.astype(jnp.bfloat16)
            else:
                out_ref[...] = xn

    return pl.pallas_call(
        body,
        out_shape=jax.ShapeDtypeStruct((b, d_sh), jnp.float32),
        in_specs=[pl.BlockSpec(memory_space=pltpu.MemorySpace.VMEM)]
        + [pl.BlockSpec(memory_space=pltpu.MemorySpace.HBM)] * 6,
        out_specs=pl.BlockSpec(memory_space=pltpu.MemorySpace.VMEM),
        scratch_shapes=[
            pltpu.VMEM((2, d_sh, h_dim), jnp.float32),
            pltpu.VMEM((2, h_dim, d_sh), jnp.float32),
            pltpu.VMEM((d_sh, h_dim), jnp.bfloat16),
            pltpu.VMEM((h_dim, d_sh), jnp.bfloat16),
            pltpu.VMEM((N_LAYERS, 2, b, h_dim), jnp.bfloat16),
            pltpu.VMEM((N_LAYERS, 2, b, h_dim), jnp.bfloat16),
            pltpu.SemaphoreType.DMA((N_LAYERS,)),
            pltpu.SemaphoreType.DMA((N_LAYERS,)),
            pltpu.SemaphoreType.DMA((N_LAYERS, 2)),
            pltpu.SemaphoreType.DMA((N_LAYERS, 2)),
        ],
        compiler_params=pltpu.CompilerParams(
            collective_id=0,
            vmem_limit_bytes=128 * 1024 * 1024,
        ),
    )(x, Win0, Wout0, Win1, Wout1, Win2, Wout2)
